# Initial kernel scaffold; baseline (speedup 1.0000x reference)
import jax
import jax.numpy as jnp
from jax.experimental import pallas as pl


def kernel(out_temp, out, coord):
    raise NotImplementedError("write your pallas kernel here")


# speedup vs baseline: 3.7482x; 3.7482x over previous
import jax
import jax.numpy as jnp
from jax.experimental import pallas as pl
from jax.experimental.pallas import tpu as pltpu


def _body(coord_ref, temp_ref, out_ref, o_ref):
    th, tw = temp_ref.shape[1], temp_ref.shape[2]
    o_ref[...] = out_ref[...]
    x0 = pl.multiple_of(coord_ref[0], 128)
    y0 = pl.multiple_of(coord_ref[2], 8)
    o_ref[0, pl.ds(y0, th), pl.ds(x0, tw)] = temp_ref[0]


def kernel(out_temp, out, coord):
    N, C, H, W = out.shape
    _, _, th, tw = out_temp.shape
    temp3 = out_temp.reshape(N * C, th, tw)
    out3 = out.reshape(N * C, H, W)

    grid_spec = pltpu.PrefetchScalarGridSpec(
        num_scalar_prefetch=1,
        grid=(N * C,),
        in_specs=[
            pl.BlockSpec((1, th, tw), lambda i, c: (i, 0, 0)),
            pl.BlockSpec((1, H, W), lambda i, c: (i, 0, 0)),
        ],
        out_specs=pl.BlockSpec((1, H, W), lambda i, c: (i, 0, 0)),
    )
    result = pl.pallas_call(
        _body,
        grid_spec=grid_spec,
        out_shape=jax.ShapeDtypeStruct((N * C, H, W), out.dtype),
    )(coord, temp3, out3)
    return result.reshape(N, C, H, W)
